# baseline (device time: 39140 ns/iter reference)
import jax
import jax.numpy as jnp
from jax import lax
from jax.experimental import pallas as pl
from jax.experimental.pallas import tpu as pltpu


def kernel(Q, K, V):
    b, q_len, h, d = Q.shape
    k_len = K.shape[1]
    scale = d ** -0.5

    def body(q_ref, k_ref, v_ref, out_ref, m_buf, l_buf, acc_buf,
             send_sems, recv_sems):
        my_x = lax.axis_index("x")
        my_y = lax.axis_index("y")
        nbr = (1 - my_x, my_y)

        qv = q_ref[:, 0, :, :].astype(jnp.float32)
        kv = k_ref[...].astype(jnp.float32)
        vv = v_ref[...].astype(jnp.float32)

        s = jnp.sum(qv[:, None, :, :] * kv, axis=-1) * scale
        m = jnp.max(s, axis=1)
        p = jnp.exp(s - m[:, None, :])
        l = jnp.sum(p, axis=1)
        acc = jnp.sum(p[:, :, :, None] * vv, axis=1)

        m_buf[0] = m
        l_buf[0] = l
        acc_buf[0] = acc

        barrier_sem = pltpu.get_barrier_semaphore()
        pl.semaphore_signal(barrier_sem, inc=1, device_id=nbr,
                            device_id_type=pl.DeviceIdType.MESH)
        pl.semaphore_wait(barrier_sem, 1)

        rdmas = []
        for i, buf in enumerate((m_buf, l_buf, acc_buf)):
            rdma = pltpu.make_async_remote_copy(
                src_ref=buf.at[0],
                dst_ref=buf.at[1],
                send_sem=send_sems.at[i],
                recv_sem=recv_sems.at[i],
                device_id=nbr,
                device_id_type=pl.DeviceIdType.MESH,
            )
            rdma.start()
            rdmas.append(rdma)
        for rdma in rdmas:
            rdma.wait()

        m0, m1 = m_buf[0], m_buf[1]
        mn = jnp.maximum(m0, m1)
        a0 = jnp.exp(m0 - mn)
        a1 = jnp.exp(m1 - mn)
        l_tot = l_buf[0] * a0 + l_buf[1] * a1
        o = (acc_buf[0] * a0[:, :, None] + acc_buf[1] * a1[:, :, None])
        o = o / l_tot[:, :, None]
        out_ref[...] = o[:, None, :, :].astype(out_ref.dtype)

    return pl.pallas_call(
        body,
        out_shape=jax.ShapeDtypeStruct((b, q_len, h, d), jnp.float32),
        in_specs=[
            pl.BlockSpec(memory_space=pltpu.VMEM),
            pl.BlockSpec(memory_space=pltpu.VMEM),
            pl.BlockSpec(memory_space=pltpu.VMEM),
        ],
        out_specs=pl.BlockSpec(memory_space=pltpu.VMEM),
        scratch_shapes=[
            pltpu.VMEM((2, b, h), jnp.float32),
            pltpu.VMEM((2, b, h), jnp.float32),
            pltpu.VMEM((2, b, h, d), jnp.float32),
            pltpu.SemaphoreType.DMA((3,)),
            pltpu.SemaphoreType.DMA((3,)),
        ],
        compiler_params=pltpu.CompilerParams(collective_id=0),
    )(Q, K, V)


# device time: 37285 ns/iter; 1.0498x vs baseline; 1.0498x over previous
import jax
import jax.numpy as jnp
from jax import lax
from jax.experimental import pallas as pl
from jax.experimental.pallas import tpu as pltpu


def kernel(Q, K, V):
    b, q_len, h, d = Q.shape
    k_len = K.shape[1]
    hd = h * d
    scale = d ** -0.5

    def body(q_ref, k_ref, v_ref, out_ref, m_buf, l_buf, acc_buf,
             send_sems, recv_sems):
        my_x = lax.axis_index("x")
        my_y = lax.axis_index("y")
        nbr = (1 - my_x, my_y)

        qv = q_ref[:, 0, :, :].astype(jnp.float32)
        qvT = jnp.transpose(qv, (0, 2, 1))
        wt = jnp.tile(qvT, (1, h, 1))
        rows = lax.broadcasted_iota(jnp.int32, (hd, h), 0)
        cols = lax.broadcasted_iota(jnp.int32, (hd, h), 1)
        wmask = (rows // d) == cols
        rows2 = lax.broadcasted_iota(jnp.int32, (h, hd), 0)
        cols2 = lax.broadcasted_iota(jnp.int32, (h, hd), 1)
        omask = rows2 == (cols2 // d)

        for bi in range(b):
            k_b = k_ref[bi].reshape(k_len, hd).astype(jnp.float32)
            v_b = v_ref[bi].reshape(k_len, hd).astype(jnp.float32)
            w_b = jnp.where(wmask, wt[bi], 0.0)
            s_b = lax.dot_general(
                w_b, k_b, (((0,), (1,)), ((), ())),
                preferred_element_type=jnp.float32,
            ) * scale
            m_b = jnp.max(s_b, axis=1, keepdims=True)
            p_b = jnp.exp(s_b - m_b)
            l_b = jnp.sum(p_b, axis=1, keepdims=True)
            r_b = jnp.dot(p_b, v_b, preferred_element_type=jnp.float32)
            o_b = jnp.sum(jnp.where(omask, r_b, 0.0), axis=0, keepdims=True)
            m_buf[0, pl.ds(bi, 1), :] = m_b.reshape(1, h)
            l_buf[0, pl.ds(bi, 1), :] = l_b.reshape(1, h)
            acc_buf[0, pl.ds(bi, 1), :] = o_b

        barrier_sem = pltpu.get_barrier_semaphore()
        pl.semaphore_signal(barrier_sem, inc=1, device_id=nbr,
                            device_id_type=pl.DeviceIdType.MESH)
        pl.semaphore_wait(barrier_sem, 1)

        rdmas = []
        for i, buf in enumerate((m_buf, l_buf, acc_buf)):
            rdma = pltpu.make_async_remote_copy(
                src_ref=buf.at[0],
                dst_ref=buf.at[1],
                send_sem=send_sems.at[i],
                recv_sem=recv_sems.at[i],
                device_id=nbr,
                device_id_type=pl.DeviceIdType.MESH,
            )
            rdma.start()
            rdmas.append(rdma)
        for rdma in rdmas:
            rdma.wait()

        m0, m1 = m_buf[0], m_buf[1]
        mn = jnp.maximum(m0, m1)
        a0 = jnp.exp(m0 - mn)
        a1 = jnp.exp(m1 - mn)
        l_tot = l_buf[0] * a0 + l_buf[1] * a1
        af0 = acc_buf[0].reshape(b, h, d)
        af1 = acc_buf[1].reshape(b, h, d)
        o = af0 * a0[:, :, None] + af1 * a1[:, :, None]
        o = o / l_tot[:, :, None]
        out_ref[...] = o[:, None, :, :].astype(out_ref.dtype)

    return pl.pallas_call(
        body,
        out_shape=jax.ShapeDtypeStruct((b, q_len, h, d), jnp.float32),
        in_specs=[
            pl.BlockSpec(memory_space=pltpu.VMEM),
            pl.BlockSpec(memory_space=pltpu.VMEM),
            pl.BlockSpec(memory_space=pltpu.VMEM),
        ],
        out_specs=pl.BlockSpec(memory_space=pltpu.VMEM),
        scratch_shapes=[
            pltpu.VMEM((2, b, h), jnp.float32),
            pltpu.VMEM((2, b, h), jnp.float32),
            pltpu.VMEM((2, b, hd), jnp.float32),
            pltpu.SemaphoreType.DMA((3,)),
            pltpu.SemaphoreType.DMA((3,)),
        ],
        compiler_params=pltpu.CompilerParams(collective_id=0),
    )(Q, K, V)


# device time: 18062 ns/iter; 2.1670x vs baseline; 2.0643x over previous
import jax
import jax.numpy as jnp
from jax import lax
from jax.experimental import pallas as pl
from jax.experimental.pallas import tpu as pltpu


def kernel(Q, K, V):
    b, q_len, h, d = Q.shape
    k_len = K.shape[1]
    hd = h * d
    scale = d ** -0.5

    Kp = lax.transpose(K, (0, 2, 3, 1))
    Vp = lax.transpose(V, (0, 2, 3, 1))

    def body(q_ref, kp_ref, vp_ref, out_ref, m_buf, l_buf, acc_buf,
             send_sems, recv_sems):
        my_x = lax.axis_index("x")
        my_y = lax.axis_index("y")
        nbr = (1 - my_x, my_y)

        selrows = lax.broadcasted_iota(jnp.int32, (h, hd), 0)
        selcols = lax.broadcasted_iota(jnp.int32, (h, hd), 1)
        selmask = (selcols // d) == selrows
        rrows = lax.broadcasted_iota(jnp.int32, (hd, h), 0)
        rcols = lax.broadcasted_iota(jnp.int32, (hd, h), 1)
        rmask = (rrows // d) == rcols

        for bi in range(b):
            q_b = q_ref[bi, 0].astype(jnp.float32)
            w2_b = jnp.where(selmask, jnp.tile(q_b, (1, h)), 0.0)
            kpf_b = kp_ref[bi].reshape(hd, k_len).astype(jnp.float32)
            vpf_b = vp_ref[bi].reshape(hd, k_len).astype(jnp.float32)
            s_b = jnp.dot(w2_b, kpf_b,
                          preferred_element_type=jnp.float32) * scale
            m_b = jnp.max(s_b, axis=1, keepdims=True)
            p_b = jnp.exp(s_b - m_b)
            l_b = jnp.sum(p_b, axis=1, keepdims=True)
            r_b = jnp.dot(vpf_b, jnp.transpose(p_b),
                          preferred_element_type=jnp.float32)
            o_b = jnp.sum(jnp.where(rmask, r_b, 0.0), axis=1,
                          keepdims=True)
            m_buf[0, :, pl.ds(bi, 1)] = m_b
            l_buf[0, :, pl.ds(bi, 1)] = l_b
            acc_buf[0, :, pl.ds(bi, 1)] = o_b

        barrier_sem = pltpu.get_barrier_semaphore()
        pl.semaphore_signal(barrier_sem, inc=1, device_id=nbr,
                            device_id_type=pl.DeviceIdType.MESH)
        pl.semaphore_wait(barrier_sem, 1)

        rdmas = []
        for i, buf in enumerate((m_buf, l_buf, acc_buf)):
            rdma = pltpu.make_async_remote_copy(
                src_ref=buf.at[0],
                dst_ref=buf.at[1],
                send_sem=send_sems.at[i],
                recv_sem=recv_sems.at[i],
                device_id=nbr,
                device_id_type=pl.DeviceIdType.MESH,
            )
            rdma.start()
            rdmas.append(rdma)
        for rdma in rdmas:
            rdma.wait()

        m0, m1 = m_buf[0], m_buf[1]
        mn = jnp.maximum(m0, m1)
        a0 = jnp.exp(m0 - mn)
        a1 = jnp.exp(m1 - mn)
        l_tot = l_buf[0] * a0 + l_buf[1] * a1
        a0e = jnp.repeat(a0, d, axis=0)
        a1e = jnp.repeat(a1, d, axis=0)
        le = jnp.repeat(l_tot, d, axis=0)
        of = (acc_buf[0] * a0e + acc_buf[1] * a1e) / le
        out_ref[...] = jnp.transpose(of).reshape(b, 1, h, d).astype(out_ref.dtype)

    return pl.pallas_call(
        body,
        out_shape=jax.ShapeDtypeStruct((b, q_len, h, d), jnp.float32),
        in_specs=[
            pl.BlockSpec(memory_space=pltpu.VMEM),
            pl.BlockSpec(memory_space=pltpu.VMEM),
            pl.BlockSpec(memory_space=pltpu.VMEM),
        ],
        out_specs=pl.BlockSpec(memory_space=pltpu.VMEM),
        scratch_shapes=[
            pltpu.VMEM((2, h, b), jnp.float32),
            pltpu.VMEM((2, h, b), jnp.float32),
            pltpu.VMEM((2, hd, b), jnp.float32),
            pltpu.SemaphoreType.DMA((3,)),
            pltpu.SemaphoreType.DMA((3,)),
        ],
        compiler_params=pltpu.CompilerParams(collective_id=0),
    )(Q, Kp, Vp)
